# baseline (speedup 1.0000x reference)
import functools

import jax
import jax.numpy as jnp
from jax import lax
from jax.experimental import pallas as pl
from jax.experimental.pallas import tpu as pltpu
from jax.experimental.pallas import tpu_sc as plsc

NUM_RELATIONS = 500
LANES = 16
NUM_CORES = 2
NUM_SUBCORES = 16
NUM_WORKERS = NUM_CORES * NUM_SUBCORES
GATHER_CHUNK = 128


@functools.partial(jax.jit, static_argnames=("batch", "dim"))
def _score(s_idx, o_idx, t_idx, entity_table, rel_table, *, batch, dim):
    b_per_w = batch // NUM_WORKERS
    n_chunks = b_per_w // GATHER_CHUNK
    mesh = plsc.VectorSubcoreMesh(core_axis_name="c", subcore_axis_name="s")

    @functools.partial(
        pl.kernel,
        out_type=jax.ShapeDtypeStruct((batch,), jnp.float32),
        mesh=mesh,
        compiler_params=pltpu.CompilerParams(needs_layout_passes=False,
                                             use_tc_tiling_on_sc=False),
        scratch_types=[
            pltpu.VMEM((b_per_w,), jnp.int32),
            pltpu.VMEM((b_per_w,), jnp.int32),
            pltpu.VMEM((b_per_w,), jnp.int32),
            pltpu.VMEM((b_per_w, dim), jnp.float32),
            pltpu.VMEM((b_per_w, dim), jnp.float32),
            pltpu.VMEM((b_per_w, dim), jnp.float32),
            pltpu.VMEM((LANES * b_per_w,), jnp.float32),
            pltpu.VMEM((b_per_w,), jnp.float32),
            pltpu.SemaphoreType.DMA,
        ],
    )
    def scorer(sidx_hbm, oidx_hbm, tidx_hbm, ent_hbm, rel_hbm, out_hbm,
               sidx_v, oidx_v, ridx_v, srows, orows, rrows, part_t, out_v,
               sem):
        wid = lax.axis_index("s") * NUM_CORES + lax.axis_index("c")
        base = wid * b_per_w

        pltpu.sync_copy(sidx_hbm.at[pl.ds(base, b_per_w)], sidx_v)
        pltpu.sync_copy(oidx_hbm.at[pl.ds(base, b_per_w)], oidx_v)
        pltpu.sync_copy(tidx_hbm.at[pl.ds(base, b_per_w)], ridx_v)

        for k in range(b_per_w // LANES):
            sl = pl.ds(k * LANES, LANES)
            ridx_v[sl] = lax.rem(ridx_v[sl],
                                 jnp.full((LANES,), NUM_RELATIONS, jnp.int32))

        copies = []
        for c in range(n_chunks):
            sl = pl.ds(c * GATHER_CHUNK, GATHER_CHUNK)
            copies.append(
                pltpu.async_copy(ent_hbm.at[sidx_v.at[sl]], srows.at[sl], sem))
            copies.append(
                pltpu.async_copy(ent_hbm.at[oidx_v.at[sl]], orows.at[sl], sem))
            copies.append(
                pltpu.async_copy(rel_hbm.at[ridx_v.at[sl]], rrows.at[sl], sem))
        for cp in copies:
            cp.wait()

        lane_rows = lax.iota(jnp.int32, LANES) * b_per_w

        def row_body(i, carry):
            acc = jnp.zeros((LANES,), jnp.float32)
            for q in range(dim // LANES):
                sl = pl.ds(q * LANES, LANES)
                acc = acc + srows[i, sl] * rrows[i, sl] * orows[i, sl]
            plsc.store_scatter(part_t, [lane_rows + i], acc)
            return carry

        lax.fori_loop(0, b_per_w, row_body, 0)

        for g in range(b_per_w // LANES):
            sl = pl.ds(g * LANES, LANES)
            acc = part_t[pl.ds(g * LANES, LANES)]
            for j in range(1, LANES):
                acc = acc + part_t[pl.ds(j * b_per_w + g * LANES, LANES)]
            out_v[sl] = acc

        pltpu.sync_copy(out_v, out_hbm.at[pl.ds(base, b_per_w)])

    return scorer(s_idx, o_idx, t_idx, entity_table, rel_table)


def kernel(triples, entity_table, rel_table):
    s_idx = triples[:, 0].astype(jnp.int32)
    o_idx = triples[:, 1].astype(jnp.int32)
    t_idx = triples[:, 2].astype(jnp.int32)
    batch = triples.shape[0]
    dim = entity_table.shape[1]
    return _score(s_idx, o_idx, t_idx, entity_table, rel_table,
                  batch=batch, dim=dim)


# speedup vs baseline: 1.6211x; 1.6211x over previous
import functools

import jax
import jax.numpy as jnp
from jax import lax
from jax.experimental import pallas as pl
from jax.experimental.pallas import tpu as pltpu
from jax.experimental.pallas import tpu_sc as plsc

NUM_RELATIONS = 500
LANES = 16
NUM_CORES = 2
NUM_SUBCORES = 16
NUM_WORKERS = NUM_CORES * NUM_SUBCORES
WAVE = 128


@functools.partial(jax.jit, static_argnames=("batch", "dim"))
def _score(s_idx, o_idx, t_idx, entity_table, rel_table, *, batch, dim):
    b_per_w = batch // NUM_WORKERS
    n_waves = b_per_w // WAVE
    n_rel = rel_table.shape[0]
    mesh = plsc.VectorSubcoreMesh(core_axis_name="c", subcore_axis_name="s")

    @functools.partial(
        pl.kernel,
        out_type=jax.ShapeDtypeStruct((batch,), jnp.float32),
        mesh=mesh,
        compiler_params=pltpu.CompilerParams(needs_layout_passes=False),
        scratch_types=[
            pltpu.VMEM((b_per_w,), jnp.int32),
            pltpu.VMEM((b_per_w,), jnp.int32),
            pltpu.VMEM((b_per_w,), jnp.int32),
            pltpu.VMEM((n_rel, dim), jnp.float32),
            pltpu.VMEM((WAVE, dim), jnp.float32),
            pltpu.VMEM((WAVE, dim), jnp.float32),
            pltpu.VMEM((LANES * b_per_w,), jnp.float32),
            pltpu.VMEM((b_per_w,), jnp.float32),
            pltpu.SemaphoreType.DMA,
        ],
    )
    def scorer(sidx_hbm, oidx_hbm, tidx_hbm, ent_hbm, rel_hbm, out_hbm,
               sidx_v, oidx_v, ridx_v, rel_v, srows, orows, part_t, out_v,
               sem):
        wid = lax.axis_index("s") * NUM_CORES + lax.axis_index("c")
        base = wid * b_per_w

        pltpu.sync_copy(sidx_hbm.at[pl.ds(base, b_per_w)], sidx_v)
        pltpu.sync_copy(oidx_hbm.at[pl.ds(base, b_per_w)], oidx_v)
        pltpu.sync_copy(tidx_hbm.at[pl.ds(base, b_per_w)], ridx_v)
        pltpu.sync_copy(rel_hbm, rel_v)

        for k in range(b_per_w // LANES):
            sl = pl.ds(k * LANES, LANES)
            ridx_v[sl] = lax.rem(ridx_v[sl],
                                 jnp.full((LANES,), NUM_RELATIONS, jnp.int32))

        lane_rows = lax.iota(jnp.int32, LANES) * b_per_w

        def fetch_group(g, w0):
            svec = sidx_v[pl.ds(w0 + g * LANES, LANES)]
            ovec = oidx_v[pl.ds(w0 + g * LANES, LANES)]
            for l in range(LANES):
                i = g * LANES + l
                pltpu.async_copy(ent_hbm.at[pl.ds(svec[l], 1)],
                                 srows.at[pl.ds(i, 1)], sem)
                pltpu.async_copy(ent_hbm.at[pl.ds(ovec[l], 1)],
                                 orows.at[pl.ds(i, 1)], sem)
            return w0

        def compute_group(g, w0):
            rvec = ridx_v[pl.ds(w0 + g * LANES, LANES)]
            for l in range(LANES):
                i = g * LANES + l
                r = rvec[l]
                acc = jnp.zeros((LANES,), jnp.float32)
                for q in range(dim // LANES):
                    sl = pl.ds(q * LANES, LANES)
                    acc = acc + srows[i, sl] * rel_v[r, sl] * orows[i, sl]
                plsc.store_scatter(part_t, [lane_rows + w0 + i], acc)
            return w0

        for w in range(n_waves):
            lax.fori_loop(0, WAVE // LANES, fetch_group, w * WAVE)
            pltpu.make_async_copy(ent_hbm.at[pl.ds(0, WAVE)], srows, sem).wait()
            pltpu.make_async_copy(ent_hbm.at[pl.ds(0, WAVE)], orows, sem).wait()
            lax.fori_loop(0, WAVE // LANES, compute_group, w * WAVE)

        for g in range(b_per_w // LANES):
            sl = pl.ds(g * LANES, LANES)
            acc = part_t[pl.ds(g * LANES, LANES)]
            for j in range(1, LANES):
                acc = acc + part_t[pl.ds(j * b_per_w + g * LANES, LANES)]
            out_v[sl] = acc

        pltpu.sync_copy(out_v, out_hbm.at[pl.ds(base, b_per_w)])

    return scorer(s_idx, o_idx, t_idx, entity_table, rel_table)


def kernel(triples, entity_table, rel_table):
    s_idx = triples[:, 0].astype(jnp.int32)
    o_idx = triples[:, 1].astype(jnp.int32)
    t_idx = triples[:, 2].astype(jnp.int32)
    batch = triples.shape[0]
    dim = entity_table.shape[1]
    return _score(s_idx, o_idx, t_idx, entity_table, rel_table,
                  batch=batch, dim=dim)
